# baseline (speedup 1.0000x reference)
import functools

import jax
import jax.numpy as jnp
from jax import lax
from jax.experimental import pallas as pl
from jax.experimental.pallas import tpu as pltpu
from jax.experimental.pallas import tpu_sc as plsc

_N = 10000
_E = 320000
_D = 128
_HD = _D // 2
_NC = 2
_NS = 16
_NW = _NC * _NS
_EPW = _E // _NW
_CHUNK = 125
_STEPS = _EPW // _CHUNK
_SROWS = _N // 10
_WSTG = 40

_mesh = plsc.VectorSubcoreMesh(core_axis_name="c", subcore_axis_name="s")
_sc_params = pltpu.CompilerParams(use_tc_tiling_on_sc=False)


@functools.partial(
    pl.kernel,
    out_type=jax.ShapeDtypeStruct((_NC * _N,), jnp.float32),
    mesh=_mesh,
    scratch_types=[
        pltpu.VMEM((_STEPS, _CHUNK), jnp.int32),
        pltpu.VMEM((128,), jnp.float32),
        pltpu.VMEM((_SROWS,), jnp.float32),
        pltpu.VMEM_SHARED((_N,), jnp.float32),
        pltpu.SemaphoreType.DMA,
    ],
)
def _sc_count(ones_hbm, dst_hbm, out_hbm, dst_v, ones_v, stage_v, acc, sem):
    c = lax.axis_index("c")
    s = lax.axis_index("s")
    wid = c * _NS + s

    pltpu.sync_copy(ones_hbm.at[pl.ds(0, 128)], ones_v)

    @pl.when(s < 10)
    def _init():
        pltpu.sync_copy(ones_hbm.at[pl.ds(0, _SROWS)], stage_v)
        pltpu.sync_copy(stage_v, acc.at[pl.ds(s * _SROWS, _SROWS)])

    pltpu.sync_copy(dst_hbm.at[wid], dst_v)
    plsc.subcore_barrier()

    def group(g, carry):
        base = g * 8
        for t in range(8):
            pltpu.async_copy(ones_v.at[pl.ds(0, _CHUNK)],
                             acc.at[dst_v.at[base + t]], sem, add=True)
        for t in range(8):
            pltpu.make_async_copy(ones_v.at[pl.ds(0, _CHUNK)],
                                  acc.at[dst_v.at[base + t]], sem).wait()
        return carry

    lax.fori_loop(0, _STEPS // 8, group, 0)
    plsc.subcore_barrier()

    @pl.when(s < 10)
    def _drain():
        pltpu.sync_copy(acc.at[pl.ds(s * _SROWS, _SROWS)], stage_v)
        pltpu.sync_copy(stage_v,
                        out_hbm.at[pl.ds(c * _N + s * _SROWS, _SROWS)])


_WSTEPS = _E // _NS // _CHUNK


@functools.partial(
    pl.kernel,
    out_type=jax.ShapeDtypeStruct((_NC, _N, _HD), jnp.float32),
    mesh=_mesh,
    scratch_types=[
        pltpu.VMEM((_WSTEPS, _CHUNK), jnp.int32),
        pltpu.VMEM((_WSTEPS, _CHUNK), jnp.int32),
        pltpu.VMEM((_CHUNK, _HD), jnp.float32),
        pltpu.VMEM((_CHUNK, _HD), jnp.float32),
        pltpu.VMEM((_CHUNK, _HD), jnp.float32),
        pltpu.VMEM((_CHUNK, _HD), jnp.float32),
        pltpu.VMEM((_WSTG, _HD), jnp.float32),
        pltpu.VMEM((_WSTG, _HD), jnp.float32),
        pltpu.VMEM_SHARED((_N, _HD), jnp.float32),
        pltpu.SemaphoreType.DMA,
        pltpu.SemaphoreType.DMA,
        pltpu.SemaphoreType.DMA,
    ],
    compiler_params=_sc_params,
)
def _sc_agg(y2n_hbm, zeros_hbm, idx_hbm, dst_hbm, out_hbm, idx_v, dst_v,
            rows0_v, rows1_v, rows2_v, rows3_v, stage_v, stage2_v, acc,
            sem0, sem1, ssem):
    c = lax.axis_index("c")
    s = lax.axis_index("s")

    @pl.when(s < 10)
    def _init():
        pltpu.sync_copy(zeros_hbm.at[pl.ds(0, _WSTG)], stage_v)
        hs = []
        for r in range(_SROWS // _WSTG):
            h = pltpu.make_async_copy(
                stage_v, acc.at[pl.ds(s * _SROWS + r * _WSTG, _WSTG)], sem0)
            h.start()
            hs.append(h)
        for h in hs:
            h.wait()

    pltpu.sync_copy(idx_hbm.at[c, s], idx_v)
    pltpu.sync_copy(dst_hbm.at[s], dst_v)

    rbufs = (rows0_v, rows1_v, rows2_v, rows3_v)
    gsems = (sem0, sem1)

    def gather(j, t):
        return pltpu.make_async_copy(y2n_hbm.at[idx_v.at[j]],
                                     rbufs[t % 4], gsems[t % 2])

    def scat_wait(j, t):
        pltpu.make_async_copy(rbufs[t % 4], acc.at[dst_v.at[j]], ssem).wait()

    gather(0, 0).start()
    gather(1, 1).start()
    plsc.subcore_barrier()

    def quad(q, carry):
        j0 = 4 * q
        for t in range(4):
            j = j0 + t
            gather(j, t).wait()

            @pl.when(j > 0)
            def _():
                scat_wait(j - 1, t + 3)

            pltpu.async_copy(rbufs[t], acc.at[dst_v.at[j]], ssem, add=True)

            @pl.when(j + 2 < _WSTEPS)
            def _():
                gather(j + 2, t + 2).start()

        return carry

    lax.fori_loop(0, _WSTEPS // 4, quad, 0)
    scat_wait(_WSTEPS - 1, 3)
    plsc.subcore_barrier()

    @pl.when(s < 10)
    def _drain():
        nst = _SROWS // _WSTG
        for r in range(nst):
            st, sm = (stage_v, sem0) if r % 2 == 0 else (stage2_v, sem1)
            if r >= 2:
                offp = pl.ds(s * _SROWS + (r - 2) * _WSTG, _WSTG)
                pltpu.make_async_copy(st, out_hbm.at[c, offp], sm).wait()
            off = pl.ds(s * _SROWS + r * _WSTG, _WSTG)
            pltpu.sync_copy(acc.at[off], st)
            pltpu.async_copy(st, out_hbm.at[c, off], sm)
        for r in (nst - 2, nst - 1):
            st, sm = (stage_v, sem0) if r % 2 == 0 else (stage2_v, sem1)
            off = pl.ds(s * _SROWS + r * _WSTG, _WSTG)
            pltpu.make_async_copy(st, out_hbm.at[c, off], sm).wait()


@functools.partial(
    pl.kernel,
    out_type=jax.ShapeDtypeStruct((_NC * _N,), jnp.float32),
    mesh=_mesh,
    scratch_types=[
        pltpu.VMEM((_STEPS, _CHUNK), jnp.int32),
        pltpu.VMEM((_STEPS, _CHUNK), jnp.int32),
        pltpu.VMEM((_CHUNK,), jnp.float32),
        pltpu.VMEM((_CHUNK,), jnp.float32),
        pltpu.VMEM((_CHUNK,), jnp.float32),
        pltpu.VMEM((_CHUNK,), jnp.float32),
        pltpu.VMEM((_SROWS,), jnp.float32),
        pltpu.VMEM_SHARED((_N,), jnp.float32),
        pltpu.SemaphoreType.DMA,
        pltpu.SemaphoreType.DMA,
        pltpu.SemaphoreType.DMA,
    ],
)
def _sc_sagg(v_hbm, src_hbm, dst_hbm, out_hbm, src_v, dst_v, vals0_v, vals1_v,
             vals2_v, vals3_v, stage_v, acc, sem0, sem1, ssem):
    c = lax.axis_index("c")
    s = lax.axis_index("s")
    wid = c * _NS + s

    @pl.when(s < 10)
    def _init():
        pltpu.sync_copy(v_hbm.at[pl.ds(s * _SROWS, _SROWS)], stage_v)
        pltpu.sync_copy(stage_v, acc.at[pl.ds(s * _SROWS, _SROWS)])

    pltpu.sync_copy(src_hbm.at[wid], src_v)
    pltpu.sync_copy(dst_hbm.at[wid], dst_v)
    plsc.subcore_barrier()

    vbufs = (vals0_v, vals1_v, vals2_v, vals3_v)

    def gather(j, t):
        return pltpu.make_async_copy(v_hbm.at[src_v.at[j]],
                                     vbufs[t % 4], (sem0, sem1)[t % 2])

    gather(0, 0).start()
    gather(1, 1).start()

    def scat_wait(j, t):
        pltpu.make_async_copy(vbufs[t % 4], acc.at[dst_v.at[j]], ssem).wait()

    def quad(q, carry):
        j0 = 4 * q
        for t in range(4):
            j = j0 + t
            gather(j, t).wait()

            @pl.when(j > 0)
            def _():
                scat_wait(j - 1, t + 3)

            pltpu.async_copy(vbufs[t], acc.at[dst_v.at[j]], ssem, add=True)

            @pl.when(j + 2 < _STEPS)
            def _():
                gather(j + 2, t + 2).start()

        return carry

    lax.fori_loop(0, _STEPS // 4, quad, 0)
    scat_wait(_STEPS - 1, 3)
    plsc.subcore_barrier()

    @pl.when(s < 10)
    def _drain():
        pltpu.sync_copy(acc.at[pl.ds(s * _SROWS, _SROWS)], stage_v)
        pltpu.sync_copy(stage_v,
                        out_hbm.at[pl.ds(c * _N + s * _SROWS, _SROWS)])


_TCB = 1000
_TCG = _N // _TCB

_spec_half3 = pl.BlockSpec((_NC, _TCB, _HD), lambda i: (0, i, 0))
_spec_col = pl.BlockSpec((_TCB, 1), lambda i: (i, 0))


def _tc_pre_body(c0, c1, x, w, dinv_ref, y_ref):
    deg = c0[...] + c1[...] - 1.0
    dinv = lax.rsqrt(deg)
    dinv_ref[...] = dinv
    y = jnp.dot(x[...], w[...], preferred_element_type=jnp.float32) * dinv
    y_ref[0] = y[:, :_HD]
    y_ref[1] = y[:, _HD:]


_tc_pre = pl.pallas_call(
    _tc_pre_body,
    grid=(_TCG,),
    in_specs=[
        _spec_col,
        _spec_col,
        pl.BlockSpec((_TCB, _D), lambda i: (i, 0)),
        pl.BlockSpec((_D, _D), lambda i: (0, 0)),
    ],
    out_specs=[_spec_col, _spec_half3],
    out_shape=[
        jax.ShapeDtypeStruct((_N, 1), jnp.float32),
        jax.ShapeDtypeStruct((2, _N, _HD), jnp.float32),
    ],
)


def _tc_mid_body(p, yf, dinv, b, w, out_ref):
    dv = dinv[...]
    agg_lo = dv * (p[0] + yf[0]) + b[:, :_HD]
    agg_hi = dv * (p[1] + yf[1]) + b[:, _HD:]
    h = jnp.maximum(jnp.concatenate([agg_lo, agg_hi], axis=1), 0.0)
    y = jnp.dot(h, w[...], preferred_element_type=jnp.float32) * dv
    out_ref[0] = y[:, :_HD]
    out_ref[1] = y[:, _HD:]


_tc_mid = pl.pallas_call(
    _tc_mid_body,
    grid=(_TCG,),
    in_specs=[
        _spec_half3,
        _spec_half3,
        _spec_col,
        pl.BlockSpec((1, _D), lambda i: (0, 0)),
        pl.BlockSpec((_D, _D), lambda i: (0, 0)),
    ],
    out_specs=_spec_half3,
    out_shape=jax.ShapeDtypeStruct((2, _N, _HD), jnp.float32),
)


def _tc_fin_body(r0, r1, zs, dinv, b3, out_ref):
    out_ref[...] = dinv[...] * (r0[...] + r1[...] - zs[...]) + b3[...]


_tc_fin = pl.pallas_call(
    _tc_fin_body,
    out_shape=jax.ShapeDtypeStruct((_N, 1), jnp.float32),
)


def kernel(x, edge_index, W1, b1, W2, b2, W3, b3):
    src = edge_index[0].reshape(_NW, _STEPS, _CHUNK)
    dst = edge_index[1].reshape(_NW, _STEPS, _CHUNK)
    srcw = edge_index[0].reshape(_NS, _WSTEPS, _CHUNK)
    dstw = edge_index[1].reshape(_NS, _WSTEPS, _CHUNK)
    src_off = jnp.stack([srcw, srcw + _N])
    ones = jnp.ones((_N,), jnp.float32)
    zeros = jnp.zeros((_N, _HD), jnp.float32)

    cnt = _sc_count(ones, dst).reshape(_NC, _N)
    dinv, y1 = _tc_pre(cnt[0].reshape(_N, 1), cnt[1].reshape(_N, 1), x, W1)

    w3p = jnp.pad(W3, ((0, 0), (0, _D - 1)))
    ws = jnp.stack([W2, w3p])
    bs = jnp.stack([b1.reshape(1, _D), b2.reshape(1, _D)])

    def body(state):
        i, yf = state
        b = lax.dynamic_index_in_dim(bs, i, 0, keepdims=False)
        w = lax.dynamic_index_in_dim(ws, i, 0, keepdims=False)
        p = _sc_agg(yf.reshape(2 * _N, _HD), zeros, src_off, dstw)
        return i + 1, _tc_mid(p, yf, dinv, b, w)

    i0 = (x[0, 0] - x[0, 0]).astype(jnp.int32)
    _, y3 = lax.while_loop(lambda st: st[0] < 2, body, (i0, y1))

    zs = y3[0, :, :1]
    r = _sc_sagg(y3[0, :, 0], src, dst).reshape(_NC, _N)
    out = _tc_fin(r[0].reshape(_N, 1), r[1].reshape(_N, 1), zs, dinv,
                  b3.reshape(1, 1))
    return out


# speedup vs baseline: 1.1193x; 1.1193x over previous
import functools

import jax
import jax.numpy as jnp
from jax import lax
from jax.experimental import pallas as pl
from jax.experimental.pallas import tpu as pltpu
from jax.experimental.pallas import tpu_sc as plsc

_N = 10000
_E = 320000
_D = 128
_HD = _D // 2
_NC = 2
_NS = 16
_NW = _NC * _NS
_EPW = _E // _NW
_CHUNK = 125
_STEPS = _EPW // _CHUNK
_SROWS = _N // 10
_WSTG = 40

_mesh = plsc.VectorSubcoreMesh(core_axis_name="c", subcore_axis_name="s")
_sc_params = pltpu.CompilerParams(use_tc_tiling_on_sc=False)


@functools.partial(
    pl.kernel,
    out_type=jax.ShapeDtypeStruct((_NC * _N,), jnp.float32),
    mesh=_mesh,
    scratch_types=[
        pltpu.VMEM((_STEPS, _CHUNK), jnp.int32),
        pltpu.VMEM((128,), jnp.float32),
        pltpu.VMEM((_SROWS,), jnp.float32),
        pltpu.VMEM_SHARED((_N,), jnp.float32),
        pltpu.SemaphoreType.DMA,
    ],
)
def _sc_count(ones_hbm, dst_hbm, out_hbm, dst_v, ones_v, stage_v, acc, sem):
    c = lax.axis_index("c")
    s = lax.axis_index("s")
    wid = c * _NS + s

    pltpu.sync_copy(ones_hbm.at[pl.ds(0, 128)], ones_v)

    @pl.when(s < 10)
    def _init():
        pltpu.sync_copy(ones_hbm.at[pl.ds(0, _SROWS)], stage_v)
        pltpu.sync_copy(stage_v, acc.at[pl.ds(s * _SROWS, _SROWS)])

    pltpu.sync_copy(dst_hbm.at[wid], dst_v)
    plsc.subcore_barrier()

    def group(g, carry):
        base = g * 8
        for t in range(8):
            pltpu.async_copy(ones_v.at[pl.ds(0, _CHUNK)],
                             acc.at[dst_v.at[base + t]], sem, add=True)
        for t in range(8):
            pltpu.make_async_copy(ones_v.at[pl.ds(0, _CHUNK)],
                                  acc.at[dst_v.at[base + t]], sem).wait()
        return carry

    lax.fori_loop(0, _STEPS // 8, group, 0)
    plsc.subcore_barrier()

    @pl.when(s < 10)
    def _drain():
        pltpu.sync_copy(acc.at[pl.ds(s * _SROWS, _SROWS)], stage_v)
        pltpu.sync_copy(stage_v,
                        out_hbm.at[pl.ds(c * _N + s * _SROWS, _SROWS)])


_WSTEPS = _E // _NS // _CHUNK


@functools.partial(
    pl.kernel,
    out_type=jax.ShapeDtypeStruct((_NC, _N, _HD), jnp.float32),
    mesh=_mesh,
    scratch_types=[
        pltpu.VMEM((_WSTEPS, _CHUNK), jnp.int32),
        pltpu.VMEM((_WSTEPS, _CHUNK), jnp.int32),
        pltpu.VMEM((_CHUNK, _HD), jnp.float32),
        pltpu.VMEM((_CHUNK, _HD), jnp.float32),
        pltpu.VMEM((_CHUNK, _HD), jnp.float32),
        pltpu.VMEM((_CHUNK, _HD), jnp.float32),
        pltpu.VMEM((_WSTG, _HD), jnp.float32),
        pltpu.VMEM((_WSTG, _HD), jnp.float32),
        pltpu.VMEM_SHARED((_N, _HD), jnp.float32),
        pltpu.SemaphoreType.DMA,
        pltpu.SemaphoreType.DMA,
        pltpu.SemaphoreType.DMA,
    ],
    compiler_params=_sc_params,
)
def _sc_agg(y2n_hbm, zeros_hbm, idx_hbm, dst_hbm, out_hbm, idx_v, dst_v,
            rows0_v, rows1_v, rows2_v, rows3_v, stage_v, stage2_v, acc,
            sem0, sem1, ssem):
    c = lax.axis_index("c")
    s = lax.axis_index("s")

    @pl.when(s < 10)
    def _init():
        pltpu.sync_copy(zeros_hbm.at[pl.ds(0, _WSTG)], stage_v)
        hs = []
        for r in range(_SROWS // _WSTG):
            h = pltpu.make_async_copy(
                stage_v, acc.at[pl.ds(s * _SROWS + r * _WSTG, _WSTG)], sem0)
            h.start()
            hs.append(h)
        for h in hs:
            h.wait()

    pltpu.sync_copy(idx_hbm.at[c, s], idx_v)
    pltpu.sync_copy(dst_hbm.at[s], dst_v)

    rbufs = (rows0_v, rows1_v, rows2_v, rows3_v)
    gsems = (sem0, sem1)

    def gather(j, t):
        return pltpu.make_async_copy(y2n_hbm.at[idx_v.at[j]],
                                     rbufs[t % 4], gsems[t % 2])

    def scat_wait(j, t):
        pltpu.make_async_copy(rbufs[t % 4], acc.at[dst_v.at[j]], ssem).wait()

    gather(0, 0).start()
    gather(1, 1).start()
    plsc.subcore_barrier()

    def quad(q, carry):
        j0 = 4 * q
        for t in range(4):
            j = j0 + t
            gather(j, t).wait()

            @pl.when(j > 0)
            def _():
                scat_wait(j - 1, t + 3)

            pltpu.async_copy(rbufs[t], acc.at[dst_v.at[j]], ssem, add=True)

            @pl.when(j + 2 < _WSTEPS)
            def _():
                gather(j + 2, t + 2).start()

        return carry

    lax.fori_loop(0, _WSTEPS // 4, quad, 0)
    scat_wait(_WSTEPS - 1, 3)
    plsc.subcore_barrier()

    @pl.when(s < 10)
    def _drain():
        nst = _SROWS // _WSTG
        for r in range(nst):
            st, sm = (stage_v, sem0) if r % 2 == 0 else (stage2_v, sem1)
            if r >= 2:
                offp = pl.ds(s * _SROWS + (r - 2) * _WSTG, _WSTG)
                pltpu.make_async_copy(st, out_hbm.at[c, offp], sm).wait()
            off = pl.ds(s * _SROWS + r * _WSTG, _WSTG)
            pltpu.sync_copy(acc.at[off], st)
            pltpu.async_copy(st, out_hbm.at[c, off], sm)
        for r in (nst - 2, nst - 1):
            st, sm = (stage_v, sem0) if r % 2 == 0 else (stage2_v, sem1)
            off = pl.ds(s * _SROWS + r * _WSTG, _WSTG)
            pltpu.make_async_copy(st, out_hbm.at[c, off], sm).wait()


@functools.partial(
    pl.kernel,
    out_type=jax.ShapeDtypeStruct((_NC * _N,), jnp.float32),
    mesh=_mesh,
    scratch_types=[
        pltpu.VMEM((_STEPS, _CHUNK), jnp.int32),
        pltpu.VMEM((_STEPS, _CHUNK), jnp.int32),
        pltpu.VMEM((_CHUNK,), jnp.float32),
        pltpu.VMEM((_CHUNK,), jnp.float32),
        pltpu.VMEM((_CHUNK,), jnp.float32),
        pltpu.VMEM((_CHUNK,), jnp.float32),
        pltpu.VMEM((_SROWS,), jnp.float32),
        pltpu.VMEM_SHARED((_N,), jnp.float32),
        pltpu.SemaphoreType.DMA,
        pltpu.SemaphoreType.DMA,
        pltpu.SemaphoreType.DMA,
    ],
)
def _sc_sagg(v_hbm, src_hbm, dst_hbm, out_hbm, src_v, dst_v, vals0_v, vals1_v,
             vals2_v, vals3_v, stage_v, acc, sem0, sem1, ssem):
    c = lax.axis_index("c")
    s = lax.axis_index("s")
    wid = c * _NS + s

    @pl.when(s < 10)
    def _init():
        pltpu.sync_copy(v_hbm.at[pl.ds(s * _SROWS, _SROWS)], stage_v)
        pltpu.sync_copy(stage_v, acc.at[pl.ds(s * _SROWS, _SROWS)])

    pltpu.sync_copy(src_hbm.at[wid], src_v)
    pltpu.sync_copy(dst_hbm.at[wid], dst_v)
    plsc.subcore_barrier()

    vbufs = (vals0_v, vals1_v, vals2_v, vals3_v)

    def gather(j, t):
        return pltpu.make_async_copy(v_hbm.at[src_v.at[j]],
                                     vbufs[t % 4], (sem0, sem1)[t % 2])

    gather(0, 0).start()
    gather(1, 1).start()

    def scat_wait(j, t):
        pltpu.make_async_copy(vbufs[t % 4], acc.at[dst_v.at[j]], ssem).wait()

    def quad(q, carry):
        j0 = 4 * q
        for t in range(4):
            j = j0 + t
            gather(j, t).wait()

            @pl.when(j > 0)
            def _():
                scat_wait(j - 1, t + 3)

            pltpu.async_copy(vbufs[t], acc.at[dst_v.at[j]], ssem, add=True)

            @pl.when(j + 2 < _STEPS)
            def _():
                gather(j + 2, t + 2).start()

        return carry

    lax.fori_loop(0, _STEPS // 4, quad, 0)
    scat_wait(_STEPS - 1, 3)
    plsc.subcore_barrier()

    @pl.when(s < 10)
    def _drain():
        pltpu.sync_copy(acc.at[pl.ds(s * _SROWS, _SROWS)], stage_v)
        pltpu.sync_copy(stage_v,
                        out_hbm.at[pl.ds(c * _N + s * _SROWS, _SROWS)])


_TCB = 1000
_TCG = _N // _TCB

_spec_half3 = pl.BlockSpec((_NC, _TCB, _HD), lambda i: (0, i, 0))
_spec_col = pl.BlockSpec((_TCB, 1), lambda i: (i, 0))
_spec_rows = pl.BlockSpec((_TCB, _D), lambda i: (i, 0))
_spec_pair = pl.BlockSpec((_NC, _TCB), lambda i: (0, i))


def _tc_pre_body(cnt, x, w, dinv_ref, y_ref):
    deg = (cnt[:, 0] + cnt[:, 1] - 1.0)[:, None]
    dinv = lax.rsqrt(deg)
    dinv_ref[...] = dinv
    y_ref[...] = jnp.dot(x[...], w[...],
                         preferred_element_type=jnp.float32) * dinv


_tc_pre = pl.pallas_call(
    _tc_pre_body,
    grid=(_TCG,),
    in_specs=[
        pl.BlockSpec((_TCB, _NC), lambda i: (i, 0)),
        _spec_rows,
        pl.BlockSpec((_D, _D), lambda i: (0, 0)),
    ],
    out_specs=[_spec_col, _spec_rows],
    out_shape=[
        jax.ShapeDtypeStruct((_N, 1), jnp.float32),
        jax.ShapeDtypeStruct((_N, _D), jnp.float32),
    ],
)


def _tc_mid_body(p, y, dinv, b, w, out_ref):
    dv = dinv[...]
    agg = dv * (jnp.concatenate([p[0], p[1]], axis=1) + y[...]) + b[...]
    h = jnp.maximum(agg, 0.0)
    out_ref[...] = jnp.dot(h, w[...], preferred_element_type=jnp.float32) * dv


_tc_mid = pl.pallas_call(
    _tc_mid_body,
    grid=(_TCG,),
    in_specs=[
        _spec_half3,
        _spec_rows,
        _spec_col,
        pl.BlockSpec((1, _D), lambda i: (0, 0)),
        pl.BlockSpec((_D, _D), lambda i: (0, 0)),
    ],
    out_specs=_spec_rows,
    out_shape=jax.ShapeDtypeStruct((_N, _D), jnp.float32),
)


def _tc_fin_body(r, zs, dinv, b3, out_ref):
    out_ref[...] = dinv[...] * ((r[0] + r[1])[:, None] - zs[...]) + b3[...]


_tc_fin = pl.pallas_call(
    _tc_fin_body,
    out_shape=jax.ShapeDtypeStruct((_N, 1), jnp.float32),
)


def kernel(x, edge_index, W1, b1, W2, b2, W3, b3):
    src = edge_index[0].reshape(_NW, _STEPS, _CHUNK)
    dst = edge_index[1].reshape(_NW, _STEPS, _CHUNK)
    srcw = edge_index[0].reshape(_NS, _WSTEPS, _CHUNK)
    dstw = edge_index[1].reshape(_NS, _WSTEPS, _CHUNK)
    src2 = jnp.stack([srcw * 2, srcw * 2 + 1])
    ones = jnp.ones((_N,), jnp.float32)
    zeros = jnp.zeros((_N, _HD), jnp.float32)

    cnt = _sc_count(ones, dst).reshape(_NC, _N).T
    dinv, y1 = _tc_pre(cnt, x, W1)

    w3p = jnp.pad(W3, ((0, 0), (0, _D - 1)))

    def body(state):
        i, y, w_cur, w_nxt, b_cur, b_nxt = state
        p = _sc_agg(y.reshape(2 * _N, _HD), zeros, src2, dstw)
        y_nxt = _tc_mid(p, y, dinv, b_cur, w_cur)
        return i + 1, y_nxt, w_nxt, w_cur, b_nxt, b_cur

    i0 = (x[0, 0] - x[0, 0]).astype(jnp.int32)
    st = (i0, y1, W2, w3p, b1.reshape(1, _D), b2.reshape(1, _D))
    y3 = lax.while_loop(lambda s: s[0] < 2, body, st)[1]

    zs = y3[:, :1]
    r = _sc_sagg(y3[:, 0], src, dst).reshape(_NC, _N)
    out = _tc_fin(r, zs, dinv, b3.reshape(1, 1))
    return out


# speedup vs baseline: 1.2056x; 1.0771x over previous
import functools

import jax
import jax.numpy as jnp
from jax import lax
from jax.experimental import pallas as pl
from jax.experimental.pallas import tpu as pltpu
from jax.experimental.pallas import tpu_sc as plsc

_N = 10000
_E = 320000
_D = 128
_HD = _D // 2
_NC = 2
_NS = 16
_NW = _NC * _NS
_EPW = _E // _NW
_CHUNK = 125
_STEPS = _EPW // _CHUNK
_SROWS = _N // 10
_WSTG = 40

_mesh = plsc.VectorSubcoreMesh(core_axis_name="c", subcore_axis_name="s")
_sc_params = pltpu.CompilerParams(use_tc_tiling_on_sc=False)


@functools.partial(
    pl.kernel,
    out_type=jax.ShapeDtypeStruct((_NC * _N,), jnp.float32),
    mesh=_mesh,
    scratch_types=[
        pltpu.VMEM((_STEPS, _CHUNK), jnp.int32),
        pltpu.VMEM((128,), jnp.float32),
        pltpu.VMEM((_SROWS,), jnp.float32),
        pltpu.VMEM_SHARED((_N,), jnp.float32),
        pltpu.SemaphoreType.DMA,
    ],
)
def _sc_count(ones_hbm, dst_hbm, out_hbm, dst_v, ones_v, stage_v, acc, sem):
    c = lax.axis_index("c")
    s = lax.axis_index("s")
    wid = c * _NS + s

    pltpu.sync_copy(ones_hbm.at[pl.ds(0, 128)], ones_v)

    @pl.when(s < 10)
    def _init():
        pltpu.sync_copy(ones_hbm.at[pl.ds(0, _SROWS)], stage_v)
        pltpu.sync_copy(stage_v, acc.at[pl.ds(s * _SROWS, _SROWS)])

    pltpu.sync_copy(dst_hbm.at[wid], dst_v)
    plsc.subcore_barrier()

    def group(g, carry):
        base = g * 8
        for t in range(8):
            pltpu.async_copy(ones_v.at[pl.ds(0, _CHUNK)],
                             acc.at[dst_v.at[base + t]], sem, add=True)
        for t in range(8):
            pltpu.make_async_copy(ones_v.at[pl.ds(0, _CHUNK)],
                                  acc.at[dst_v.at[base + t]], sem).wait()
        return carry

    lax.fori_loop(0, _STEPS // 8, group, 0)
    plsc.subcore_barrier()

    @pl.when(s < 10)
    def _drain():
        pltpu.sync_copy(acc.at[pl.ds(s * _SROWS, _SROWS)], stage_v)
        pltpu.sync_copy(stage_v,
                        out_hbm.at[pl.ds(c * _N + s * _SROWS, _SROWS)])


_WSTEPS = _E // _NS // _CHUNK


@functools.partial(
    pl.kernel,
    out_type=jax.ShapeDtypeStruct((_NC, _N, _HD), jnp.float32),
    mesh=_mesh,
    scratch_types=[
        pltpu.VMEM((_WSTEPS, _CHUNK), jnp.int32),
        pltpu.VMEM((_WSTEPS, _CHUNK), jnp.int32),
        pltpu.VMEM((_CHUNK, _HD), jnp.float32),
        pltpu.VMEM((_CHUNK, _HD), jnp.float32),
        pltpu.VMEM((_CHUNK, _HD), jnp.float32),
        pltpu.VMEM((_CHUNK, _HD), jnp.float32),
        pltpu.VMEM((_WSTG, _HD), jnp.float32),
        pltpu.VMEM((_WSTG, _HD), jnp.float32),
        pltpu.VMEM_SHARED((_N, _HD), jnp.float32),
        pltpu.SemaphoreType.DMA,
        pltpu.SemaphoreType.DMA,
        pltpu.SemaphoreType.DMA,
        pltpu.SemaphoreType.DMA,
        pltpu.SemaphoreType.DMA,
    ],
    compiler_params=_sc_params,
)
def _sc_agg(y2n_hbm, zeros_hbm, idx_hbm, dst_hbm, out_hbm, idx_v, dst_v,
            rows0_v, rows1_v, rows2_v, rows3_v, stage_v, stage2_v, acc,
            sem0, sem1, sem2, sem3, ssem):
    c = lax.axis_index("c")
    s = lax.axis_index("s")

    @pl.when(s < 10)
    def _init():
        pltpu.sync_copy(zeros_hbm.at[pl.ds(0, _WSTG)], stage_v)
        hs = []
        for r in range(_SROWS // _WSTG):
            h = pltpu.make_async_copy(
                stage_v, acc.at[pl.ds(s * _SROWS + r * _WSTG, _WSTG)], sem0)
            h.start()
            hs.append(h)
        for h in hs:
            h.wait()

    pltpu.sync_copy(idx_hbm.at[c, s], idx_v)
    pltpu.sync_copy(dst_hbm.at[s], dst_v)

    rbufs = (rows0_v, rows1_v, rows2_v, rows3_v)
    gsems = (sem0, sem1, sem2, sem3)

    def gather(j, t):
        return pltpu.make_async_copy(y2n_hbm.at[idx_v.at[j]],
                                     rbufs[t % 4], gsems[t % 4])

    def scat_wait(j, t):
        pltpu.make_async_copy(rbufs[t % 4], acc.at[dst_v.at[j]], ssem).wait()

    gather(0, 0).start()
    gather(1, 1).start()
    gather(2, 2).start()
    plsc.subcore_barrier()

    def quad(q, carry):
        j0 = 4 * q
        for t in range(4):
            j = j0 + t
            gather(j, t).wait()

            @pl.when(j > 0)
            def _():
                scat_wait(j - 1, t + 3)

            pltpu.async_copy(rbufs[t], acc.at[dst_v.at[j]], ssem, add=True)

            @pl.when(j + 3 < _WSTEPS)
            def _():
                gather(j + 3, t + 3).start()

        return carry

    lax.fori_loop(0, _WSTEPS // 4, quad, 0)
    scat_wait(_WSTEPS - 1, 3)
    plsc.subcore_barrier()

    @pl.when(s < 10)
    def _drain():
        nst = _SROWS // _WSTG
        for r in range(nst):
            st, sm = (stage_v, sem0) if r % 2 == 0 else (stage2_v, sem1)
            if r >= 2:
                offp = pl.ds(s * _SROWS + (r - 2) * _WSTG, _WSTG)
                pltpu.make_async_copy(st, out_hbm.at[c, offp], sm).wait()
            off = pl.ds(s * _SROWS + r * _WSTG, _WSTG)
            pltpu.sync_copy(acc.at[off], st)
            pltpu.async_copy(st, out_hbm.at[c, off], sm)
        for r in (nst - 2, nst - 1):
            st, sm = (stage_v, sem0) if r % 2 == 0 else (stage2_v, sem1)
            off = pl.ds(s * _SROWS + r * _WSTG, _WSTG)
            pltpu.make_async_copy(st, out_hbm.at[c, off], sm).wait()


@functools.partial(
    pl.kernel,
    out_type=jax.ShapeDtypeStruct((_NC * _N,), jnp.float32),
    mesh=_mesh,
    scratch_types=[
        pltpu.VMEM((_STEPS, _CHUNK), jnp.int32),
        pltpu.VMEM((_STEPS, _CHUNK), jnp.int32),
        pltpu.VMEM((_CHUNK,), jnp.float32),
        pltpu.VMEM((_CHUNK,), jnp.float32),
        pltpu.VMEM((_CHUNK,), jnp.float32),
        pltpu.VMEM((_CHUNK,), jnp.float32),
        pltpu.VMEM((_SROWS,), jnp.float32),
        pltpu.VMEM_SHARED((_N,), jnp.float32),
        pltpu.SemaphoreType.DMA,
        pltpu.SemaphoreType.DMA,
    ],
)
def _sc_sagg(v_hbm, src_hbm, dst_hbm, out_hbm, src_v, dst_v, vals0_v, vals1_v,
             vals2_v, vals3_v, stage_v, acc, sem0, sem1):
    c = lax.axis_index("c")
    s = lax.axis_index("s")
    wid = c * _NS + s

    @pl.when(s < 10)
    def _init():
        pltpu.sync_copy(v_hbm.at[pl.ds(s * _SROWS, _SROWS)], stage_v)
        pltpu.sync_copy(stage_v, acc.at[pl.ds(s * _SROWS, _SROWS)])

    pltpu.sync_copy(src_hbm.at[wid], src_v)
    pltpu.sync_copy(dst_hbm.at[wid], dst_v)
    plsc.subcore_barrier()

    vbufs = (vals0_v, vals1_v, vals2_v, vals3_v)

    def gather(j, t):
        return pltpu.make_async_copy(v_hbm.at[src_v.at[j]],
                                     vbufs[t % 4], (sem0, sem1)[t % 2])

    gather(0, 0).start()
    gather(1, 1).start()

    def quad(q, carry):
        j0 = 4 * q
        for t in range(4):
            j = j0 + t
            gather(j, t).wait()
            pltpu.sync_copy(vbufs[t], acc.at[dst_v.at[j]], add=True)

            @pl.when(j + 2 < _STEPS)
            def _():
                gather(j + 2, t + 2).start()

        return carry

    lax.fori_loop(0, _STEPS // 4, quad, 0)
    plsc.subcore_barrier()

    @pl.when(s < 10)
    def _drain():
        pltpu.sync_copy(acc.at[pl.ds(s * _SROWS, _SROWS)], stage_v)
        pltpu.sync_copy(stage_v,
                        out_hbm.at[pl.ds(c * _N + s * _SROWS, _SROWS)])


_TCB = 1000
_TCG = _N // _TCB

_spec_half3 = pl.BlockSpec((_NC, _TCB, _HD), lambda i: (0, i, 0))
_spec_col = pl.BlockSpec((_TCB, 1), lambda i: (i, 0))
_spec_rows = pl.BlockSpec((_TCB, _D), lambda i: (i, 0))
_spec_pair = pl.BlockSpec((_NC, _TCB), lambda i: (0, i))


def _tc_pre_body(cnt, x, w, dinv_ref, y_ref):
    deg = (cnt[:, 0] + cnt[:, 1] - 1.0)[:, None]
    dinv = lax.rsqrt(deg)
    dinv_ref[...] = dinv
    y_ref[...] = jnp.dot(x[...], w[...],
                         preferred_element_type=jnp.float32) * dinv


_tc_pre = pl.pallas_call(
    _tc_pre_body,
    grid=(_TCG,),
    in_specs=[
        pl.BlockSpec((_TCB, _NC), lambda i: (i, 0)),
        _spec_rows,
        pl.BlockSpec((_D, _D), lambda i: (0, 0)),
    ],
    out_specs=[_spec_col, _spec_rows],
    out_shape=[
        jax.ShapeDtypeStruct((_N, 1), jnp.float32),
        jax.ShapeDtypeStruct((_N, _D), jnp.float32),
    ],
)


def _tc_mid_body(p, y, dinv, b, w, out_ref):
    dv = dinv[...]
    agg = dv * (jnp.concatenate([p[0], p[1]], axis=1) + y[...]) + b[...]
    h = jnp.maximum(agg, 0.0)
    out_ref[...] = jnp.dot(h, w[...], preferred_element_type=jnp.float32) * dv


_tc_mid = pl.pallas_call(
    _tc_mid_body,
    grid=(_TCG,),
    in_specs=[
        _spec_half3,
        _spec_rows,
        _spec_col,
        pl.BlockSpec((1, _D), lambda i: (0, 0)),
        pl.BlockSpec((_D, _D), lambda i: (0, 0)),
    ],
    out_specs=_spec_rows,
    out_shape=jax.ShapeDtypeStruct((_N, _D), jnp.float32),
)


def _tc_fin_body(r, zs, dinv, b3, out_ref):
    out_ref[...] = dinv[...] * ((r[0] + r[1])[:, None] - zs[...]) + b3[...]


_tc_fin = pl.pallas_call(
    _tc_fin_body,
    out_shape=jax.ShapeDtypeStruct((_N, 1), jnp.float32),
)


def kernel(x, edge_index, W1, b1, W2, b2, W3, b3):
    src = edge_index[0].reshape(_NW, _STEPS, _CHUNK)
    dst = edge_index[1].reshape(_NW, _STEPS, _CHUNK)
    srcw = edge_index[0].reshape(_NS, _WSTEPS, _CHUNK)
    dstw = edge_index[1].reshape(_NS, _WSTEPS, _CHUNK)
    src2 = jnp.stack([srcw * 2, srcw * 2 + 1])
    ones = jnp.ones((_N,), jnp.float32)
    zeros = jnp.zeros((_N, _HD), jnp.float32)

    cnt = _sc_count(ones, dst).reshape(_NC, _N).T
    dinv, y1 = _tc_pre(cnt, x, W1)

    w3p = jnp.pad(W3, ((0, 0), (0, _D - 1)))

    def body(state):
        i, y, w_cur, w_nxt, b_cur, b_nxt = state
        p = _sc_agg(y.reshape(2 * _N, _HD), zeros, src2, dstw)
        y_nxt = _tc_mid(p, y, dinv, b_cur, w_cur)
        return i + 1, y_nxt, w_nxt, w_cur, b_nxt, b_cur

    i0 = (x[0, 0] - x[0, 0]).astype(jnp.int32)
    st = (i0, y1, W2, w3p, b1.reshape(1, _D), b2.reshape(1, _D))
    y3 = lax.while_loop(lambda s: s[0] < 2, body, st)[1]

    zs = y3[:, :1]
    r = _sc_sagg(y3[:, 0], src, dst).reshape(_NC, _N)
    out = _tc_fin(r, zs, dinv, b3.reshape(1, 1))
    return out


# speedup vs baseline: 1.2059x; 1.0003x over previous
import functools

import jax
import jax.numpy as jnp
from jax import lax
from jax.experimental import pallas as pl
from jax.experimental.pallas import tpu as pltpu
from jax.experimental.pallas import tpu_sc as plsc

_N = 10000
_E = 320000
_D = 128
_HD = _D // 2
_NC = 2
_NS = 16
_NW = _NC * _NS
_EPW = _E // _NW
_CHUNK = 125
_STEPS = _EPW // _CHUNK
_SROWS = _N // 10
_WSTG = 40

_mesh = plsc.VectorSubcoreMesh(core_axis_name="c", subcore_axis_name="s")
_sc_params = pltpu.CompilerParams(use_tc_tiling_on_sc=False)


@functools.partial(
    pl.kernel,
    out_type=jax.ShapeDtypeStruct((_NC * _N,), jnp.float32),
    mesh=_mesh,
    scratch_types=[
        pltpu.VMEM((_STEPS, _CHUNK), jnp.int32),
        pltpu.VMEM((128,), jnp.float32),
        pltpu.VMEM((_SROWS,), jnp.float32),
        pltpu.VMEM_SHARED((_N,), jnp.float32),
        pltpu.SemaphoreType.DMA,
    ],
)
def _sc_count(ones_hbm, dst_hbm, out_hbm, dst_v, ones_v, stage_v, acc, sem):
    c = lax.axis_index("c")
    s = lax.axis_index("s")
    wid = c * _NS + s

    pltpu.sync_copy(ones_hbm.at[pl.ds(0, 128)], ones_v)

    @pl.when(s < 10)
    def _init():
        pltpu.sync_copy(ones_hbm.at[pl.ds(0, _SROWS)], stage_v)
        pltpu.sync_copy(stage_v, acc.at[pl.ds(s * _SROWS, _SROWS)])

    pltpu.sync_copy(dst_hbm.at[wid], dst_v)
    plsc.subcore_barrier()

    def group(g, carry):
        base = g * 8
        for t in range(8):
            pltpu.async_copy(ones_v.at[pl.ds(0, _CHUNK)],
                             acc.at[dst_v.at[base + t]], sem, add=True)
        for t in range(8):
            pltpu.make_async_copy(ones_v.at[pl.ds(0, _CHUNK)],
                                  acc.at[dst_v.at[base + t]], sem).wait()
        return carry

    lax.fori_loop(0, _STEPS // 8, group, 0)
    plsc.subcore_barrier()

    @pl.when(s < 10)
    def _drain():
        pltpu.sync_copy(acc.at[pl.ds(s * _SROWS, _SROWS)], stage_v)
        pltpu.sync_copy(stage_v,
                        out_hbm.at[pl.ds(c * _N + s * _SROWS, _SROWS)])


_WSTEPS = _E // _NS // _CHUNK


@functools.partial(
    pl.kernel,
    out_type=jax.ShapeDtypeStruct((_NC, _N, _HD), jnp.float32),
    mesh=_mesh,
    scratch_types=[
        pltpu.VMEM((_WSTEPS, _CHUNK), jnp.int32),
        pltpu.VMEM((_WSTEPS, _CHUNK), jnp.int32),
        pltpu.VMEM((_CHUNK, _HD), jnp.float32),
        pltpu.VMEM((_CHUNK, _HD), jnp.float32),
        pltpu.VMEM((_CHUNK, _HD), jnp.float32),
        pltpu.VMEM((_CHUNK, _HD), jnp.float32),
        pltpu.VMEM((_WSTG, _HD), jnp.float32),
        pltpu.VMEM((_WSTG, _HD), jnp.float32),
        pltpu.VMEM_SHARED((_N, _HD), jnp.float32),
        pltpu.SemaphoreType.DMA,
        pltpu.SemaphoreType.DMA,
        pltpu.SemaphoreType.DMA,
        pltpu.SemaphoreType.DMA,
        pltpu.SemaphoreType.DMA,
    ],
    compiler_params=_sc_params,
)
def _sc_agg(y2n_hbm, zeros_hbm, idx_hbm, dst_hbm, out_hbm, idx_v, dst_v,
            rows0_v, rows1_v, rows2_v, rows3_v, stage_v, stage2_v, acc,
            sem0, sem1, sem2, sem3, ssem):
    c = lax.axis_index("c")
    s = lax.axis_index("s")

    @pl.when(s < 10)
    def _init():
        pltpu.sync_copy(zeros_hbm.at[pl.ds(0, _WSTG)], stage_v)
        hs = []
        for r in range(_SROWS // _WSTG):
            h = pltpu.make_async_copy(
                stage_v, acc.at[pl.ds(s * _SROWS + r * _WSTG, _WSTG)], sem0)
            h.start()
            hs.append(h)
        for h in hs:
            h.wait()

    pltpu.sync_copy(idx_hbm.at[c, s], idx_v)
    pltpu.sync_copy(dst_hbm.at[s], dst_v)

    rbufs = (rows0_v, rows1_v, rows2_v, rows3_v)
    gsems = (sem0, sem1, sem2, sem3)

    def gather(j, t):
        return pltpu.make_async_copy(y2n_hbm.at[idx_v.at[j]],
                                     rbufs[t % 4], gsems[t % 4])

    def scat_wait(j, t):
        pltpu.make_async_copy(rbufs[t % 4], acc.at[dst_v.at[j]], ssem).wait()

    gather(0, 0).start()
    gather(1, 1).start()
    gather(2, 2).start()
    plsc.subcore_barrier()

    def quad(q, carry):
        j0 = 4 * q
        for t in range(4):
            j = j0 + t
            gather(j, t).wait()

            @pl.when(j > 0)
            def _():
                scat_wait(j - 1, t + 3)

            pltpu.async_copy(rbufs[t], acc.at[dst_v.at[j]], ssem, add=True)

            @pl.when(j + 3 < _WSTEPS)
            def _():
                gather(j + 3, t + 3).start()

        return carry

    lax.fori_loop(0, _WSTEPS // 4, quad, 0)
    scat_wait(_WSTEPS - 1, 3)
    plsc.subcore_barrier()

    @pl.when(s < 10)
    def _drain():
        nst = _SROWS // _WSTG
        for r in range(nst):
            st, sm = (stage_v, sem0) if r % 2 == 0 else (stage2_v, sem1)
            if r >= 2:
                offp = pl.ds(s * _SROWS + (r - 2) * _WSTG, _WSTG)
                pltpu.make_async_copy(st, out_hbm.at[c, offp], sm).wait()
            off = pl.ds(s * _SROWS + r * _WSTG, _WSTG)
            pltpu.sync_copy(acc.at[off], st)
            pltpu.async_copy(st, out_hbm.at[c, off], sm)
        for r in (nst - 2, nst - 1):
            st, sm = (stage_v, sem0) if r % 2 == 0 else (stage2_v, sem1)
            off = pl.ds(s * _SROWS + r * _WSTG, _WSTG)
            pltpu.make_async_copy(st, out_hbm.at[c, off], sm).wait()


@functools.partial(
    pl.kernel,
    out_type=jax.ShapeDtypeStruct((_NC * _N,), jnp.float32),
    mesh=_mesh,
    scratch_types=[
        pltpu.VMEM((_STEPS, _CHUNK), jnp.int32),
        pltpu.VMEM((_STEPS, _CHUNK), jnp.int32),
        pltpu.VMEM((_CHUNK,), jnp.float32),
        pltpu.VMEM((_CHUNK,), jnp.float32),
        pltpu.VMEM((_CHUNK,), jnp.float32),
        pltpu.VMEM((_CHUNK,), jnp.float32),
        pltpu.VMEM((_SROWS,), jnp.float32),
        pltpu.VMEM_SHARED((_N,), jnp.float32),
        pltpu.SemaphoreType.DMA,
        pltpu.SemaphoreType.DMA,
    ],
)
def _sc_sagg(v_hbm, src_hbm, dst_hbm, out_hbm, src_v, dst_v, vals0_v, vals1_v,
             vals2_v, vals3_v, stage_v, acc, sem0, sem1):
    c = lax.axis_index("c")
    s = lax.axis_index("s")
    wid = c * _NS + s

    @pl.when(s < 10)
    def _init():
        pltpu.sync_copy(v_hbm.at[pl.ds(s * _SROWS, _SROWS)], stage_v)
        pltpu.sync_copy(stage_v, acc.at[pl.ds(s * _SROWS, _SROWS)])

    pltpu.sync_copy(src_hbm.at[wid], src_v)
    pltpu.sync_copy(dst_hbm.at[wid], dst_v)
    plsc.subcore_barrier()

    vbufs = (vals0_v, vals1_v, vals2_v, vals3_v)

    def gather(j, t):
        return pltpu.make_async_copy(v_hbm.at[src_v.at[j]],
                                     vbufs[t % 4], (sem0, sem1)[t % 2])

    gather(0, 0).start()
    gather(1, 1).start()

    def quad(q, carry):
        j0 = 4 * q
        for t in range(4):
            j = j0 + t
            gather(j, t).wait()
            pltpu.sync_copy(vbufs[t], acc.at[dst_v.at[j]], add=True)

            @pl.when(j + 2 < _STEPS)
            def _():
                gather(j + 2, t + 2).start()

        return carry

    lax.fori_loop(0, _STEPS // 4, quad, 0)
    plsc.subcore_barrier()

    @pl.when(s < 10)
    def _drain():
        pltpu.sync_copy(acc.at[pl.ds(s * _SROWS, _SROWS)], stage_v)
        pltpu.sync_copy(stage_v,
                        out_hbm.at[pl.ds(c * _N + s * _SROWS, _SROWS)])


_TCB = 2000
_TCG = _N // _TCB

_spec_half3 = pl.BlockSpec((_NC, _TCB, _HD), lambda i: (0, i, 0))
_spec_col = pl.BlockSpec((_TCB, 1), lambda i: (i, 0))
_spec_rows = pl.BlockSpec((_TCB, _D), lambda i: (i, 0))
_spec_pair = pl.BlockSpec((_NC, _TCB), lambda i: (0, i))


def _tc_pre_body(cnt, x, w, dinv_ref, y_ref):
    deg = (cnt[:, 0] + cnt[:, 1] - 1.0)[:, None]
    dinv = lax.rsqrt(deg)
    dinv_ref[...] = dinv
    y_ref[...] = jnp.dot(x[...], w[...],
                         preferred_element_type=jnp.float32) * dinv


_tc_pre = pl.pallas_call(
    _tc_pre_body,
    grid=(_TCG,),
    in_specs=[
        pl.BlockSpec((_TCB, _NC), lambda i: (i, 0)),
        _spec_rows,
        pl.BlockSpec((_D, _D), lambda i: (0, 0)),
    ],
    out_specs=[_spec_col, _spec_rows],
    out_shape=[
        jax.ShapeDtypeStruct((_N, 1), jnp.float32),
        jax.ShapeDtypeStruct((_N, _D), jnp.float32),
    ],
)


def _tc_mid_body(p, y, dinv, b, w, out_ref):
    dv = dinv[...]
    agg = dv * (jnp.concatenate([p[0], p[1]], axis=1) + y[...]) + b[...]
    h = jnp.maximum(agg, 0.0)
    out_ref[...] = jnp.dot(h, w[...], preferred_element_type=jnp.float32) * dv


_tc_mid = pl.pallas_call(
    _tc_mid_body,
    grid=(_TCG,),
    in_specs=[
        _spec_half3,
        _spec_rows,
        _spec_col,
        pl.BlockSpec((1, _D), lambda i: (0, 0)),
        pl.BlockSpec((_D, _D), lambda i: (0, 0)),
    ],
    out_specs=_spec_rows,
    out_shape=jax.ShapeDtypeStruct((_N, _D), jnp.float32),
)


def _tc_fin_body(r, zs, dinv, b3, out_ref):
    out_ref[...] = dinv[...] * ((r[0] + r[1])[:, None] - zs[...]) + b3[...]


_tc_fin = pl.pallas_call(
    _tc_fin_body,
    out_shape=jax.ShapeDtypeStruct((_N, 1), jnp.float32),
)


def kernel(x, edge_index, W1, b1, W2, b2, W3, b3):
    src = edge_index[0].reshape(_NW, _STEPS, _CHUNK)
    dst = edge_index[1].reshape(_NW, _STEPS, _CHUNK)
    srcw = edge_index[0].reshape(_NS, _WSTEPS, _CHUNK)
    dstw = edge_index[1].reshape(_NS, _WSTEPS, _CHUNK)
    src2 = jnp.stack([srcw * 2, srcw * 2 + 1])
    ones = jnp.ones((_N,), jnp.float32)
    zeros = jnp.zeros((_N, _HD), jnp.float32)

    cnt = _sc_count(ones, dst).reshape(_NC, _N).T
    dinv, y1 = _tc_pre(cnt, x, W1)

    w3p = jnp.pad(W3, ((0, 0), (0, _D - 1)))

    def body(state):
        i, y, w_cur, w_nxt, b_cur, b_nxt = state
        p = _sc_agg(y.reshape(2 * _N, _HD), zeros, src2, dstw)
        y_nxt = _tc_mid(p, y, dinv, b_cur, w_cur)
        return i + 1, y_nxt, w_nxt, w_cur, b_nxt, b_cur

    i0 = (x[0, 0] - x[0, 0]).astype(jnp.int32)
    st = (i0, y1, W2, w3p, b1.reshape(1, _D), b2.reshape(1, _D))
    y3 = lax.while_loop(lambda s: s[0] < 2, body, st)[1]

    zs = y3[:, :1]
    r = _sc_sagg(y3[:, 0], src, dst).reshape(_NC, _N)
    out = _tc_fin(r, zs, dinv, b3.reshape(1, 1))
    return out


# speedup vs baseline: 1.2062x; 1.0002x over previous
import functools

import jax
import jax.numpy as jnp
from jax import lax
from jax.experimental import pallas as pl
from jax.experimental.pallas import tpu as pltpu
from jax.experimental.pallas import tpu_sc as plsc

_N = 10000
_E = 320000
_D = 128
_HD = _D // 2
_NC = 2
_NS = 16
_NW = _NC * _NS
_EPW = _E // _NW
_CHUNK = 125
_STEPS = _EPW // _CHUNK
_SROWS = _N // 10
_WSTG = 40

_mesh = plsc.VectorSubcoreMesh(core_axis_name="c", subcore_axis_name="s")
_sc_params = pltpu.CompilerParams(use_tc_tiling_on_sc=False)


@functools.partial(
    pl.kernel,
    out_type=jax.ShapeDtypeStruct((_NC * _N,), jnp.float32),
    mesh=_mesh,
    scratch_types=[
        pltpu.VMEM((_STEPS, _CHUNK), jnp.int32),
        pltpu.VMEM((128,), jnp.float32),
        pltpu.VMEM((_SROWS,), jnp.float32),
        pltpu.VMEM_SHARED((_N,), jnp.float32),
        pltpu.SemaphoreType.DMA,
    ],
)
def _sc_count(ones_hbm, dst_hbm, out_hbm, dst_v, ones_v, stage_v, acc, sem):
    c = lax.axis_index("c")
    s = lax.axis_index("s")
    wid = c * _NS + s

    pltpu.sync_copy(ones_hbm.at[pl.ds(0, 128)], ones_v)

    @pl.when(s < 10)
    def _init():
        pltpu.sync_copy(ones_hbm.at[pl.ds(0, _SROWS)], stage_v)
        pltpu.sync_copy(stage_v, acc.at[pl.ds(s * _SROWS, _SROWS)])

    pltpu.sync_copy(dst_hbm.at[wid], dst_v)
    plsc.subcore_barrier()

    def group(g, carry):
        base = g * 8
        for t in range(8):
            pltpu.async_copy(ones_v.at[pl.ds(0, _CHUNK)],
                             acc.at[dst_v.at[base + t]], sem, add=True)
        for t in range(8):
            pltpu.make_async_copy(ones_v.at[pl.ds(0, _CHUNK)],
                                  acc.at[dst_v.at[base + t]], sem).wait()
        return carry

    lax.fori_loop(0, _STEPS // 8, group, 0)
    plsc.subcore_barrier()

    @pl.when(s < 10)
    def _drain():
        pltpu.sync_copy(acc.at[pl.ds(s * _SROWS, _SROWS)], stage_v)
        pltpu.sync_copy(stage_v,
                        out_hbm.at[pl.ds(c * _N + s * _SROWS, _SROWS)])


_WSTEPS = _E // _NS // _CHUNK


@functools.partial(
    pl.kernel,
    out_type=jax.ShapeDtypeStruct((_NC, _N, _HD), jnp.float32),
    mesh=_mesh,
    scratch_types=[
        pltpu.VMEM((_WSTEPS, _CHUNK), jnp.int32),
        pltpu.VMEM((_WSTEPS, _CHUNK), jnp.int32),
        pltpu.VMEM((_CHUNK, _HD), jnp.float32),
        pltpu.VMEM((_CHUNK, _HD), jnp.float32),
        pltpu.VMEM((_CHUNK, _HD), jnp.float32),
        pltpu.VMEM((_CHUNK, _HD), jnp.float32),
        pltpu.VMEM((_WSTG, _HD), jnp.float32),
        pltpu.VMEM((_WSTG, _HD), jnp.float32),
        pltpu.VMEM_SHARED((_N, _HD), jnp.float32),
        pltpu.SemaphoreType.DMA,
        pltpu.SemaphoreType.DMA,
        pltpu.SemaphoreType.DMA,
        pltpu.SemaphoreType.DMA,
        pltpu.SemaphoreType.DMA,
    ],
    compiler_params=_sc_params,
)
def _sc_agg(y2n_hbm, zeros_hbm, idx_hbm, dst_hbm, out_hbm, idx_v, dst_v,
            rows0_v, rows1_v, rows2_v, rows3_v, stage_v, stage2_v, acc,
            sem0, sem1, sem2, sem3, ssem):
    c = lax.axis_index("c")
    s = lax.axis_index("s")

    @pl.when(s < 10)
    def _init():
        pltpu.sync_copy(zeros_hbm.at[pl.ds(0, _WSTG)], stage_v)
        hs = []
        for r in range(_SROWS // _WSTG):
            h = pltpu.make_async_copy(
                stage_v, acc.at[pl.ds(s * _SROWS + r * _WSTG, _WSTG)], sem0)
            h.start()
            hs.append(h)
        for h in hs:
            h.wait()

    pltpu.sync_copy(idx_hbm.at[c, s], idx_v)
    pltpu.sync_copy(dst_hbm.at[s], dst_v)

    rbufs = (rows0_v, rows1_v, rows2_v, rows3_v)
    gsems = (sem0, sem1, sem2, sem3)

    def gather(j, t):
        return pltpu.make_async_copy(y2n_hbm.at[idx_v.at[j]],
                                     rbufs[t % 4], gsems[t % 4])

    def scat_wait(j, t):
        pltpu.make_async_copy(rbufs[t % 4], acc.at[dst_v.at[j]], ssem).wait()

    gather(0, 0).start()
    gather(1, 1).start()
    gather(2, 2).start()
    plsc.subcore_barrier()

    def quad(q, carry):
        j0 = 4 * q
        for t in range(4):
            j = j0 + t
            gather(j, t).wait()

            @pl.when(j > 0)
            def _():
                scat_wait(j - 1, t + 3)

            pltpu.async_copy(rbufs[t], acc.at[dst_v.at[j]], ssem, add=True)

            @pl.when(j + 3 < _WSTEPS)
            def _():
                gather(j + 3, t + 3).start()

        return carry

    lax.fori_loop(0, _WSTEPS // 4, quad, 0)
    scat_wait(_WSTEPS - 1, 3)
    plsc.subcore_barrier()

    @pl.when(s < 10)
    def _drain():
        nst = _SROWS // _WSTG
        for r in range(nst):
            st, sm = (stage_v, sem0) if r % 2 == 0 else (stage2_v, sem1)
            if r >= 2:
                offp = pl.ds(s * _SROWS + (r - 2) * _WSTG, _WSTG)
                pltpu.make_async_copy(st, out_hbm.at[c, offp], sm).wait()
            off = pl.ds(s * _SROWS + r * _WSTG, _WSTG)
            pltpu.sync_copy(acc.at[off], st)
            pltpu.async_copy(st, out_hbm.at[c, off], sm)
        for r in (nst - 2, nst - 1):
            st, sm = (stage_v, sem0) if r % 2 == 0 else (stage2_v, sem1)
            off = pl.ds(s * _SROWS + r * _WSTG, _WSTG)
            pltpu.make_async_copy(st, out_hbm.at[c, off], sm).wait()


@functools.partial(
    pl.kernel,
    out_type=jax.ShapeDtypeStruct((_NC * _N,), jnp.float32),
    mesh=_mesh,
    scratch_types=[
        pltpu.VMEM((_STEPS, _CHUNK), jnp.int32),
        pltpu.VMEM((_STEPS, _CHUNK), jnp.int32),
        pltpu.VMEM((_CHUNK,), jnp.float32),
        pltpu.VMEM((_CHUNK,), jnp.float32),
        pltpu.VMEM((_CHUNK,), jnp.float32),
        pltpu.VMEM((_CHUNK,), jnp.float32),
        pltpu.VMEM((_SROWS,), jnp.float32),
        pltpu.VMEM_SHARED((_N,), jnp.float32),
        pltpu.SemaphoreType.DMA,
        pltpu.SemaphoreType.DMA,
    ],
)
def _sc_sagg(v_hbm, src_hbm, dst_hbm, out_hbm, src_v, dst_v, vals0_v, vals1_v,
             vals2_v, vals3_v, stage_v, acc, sem0, sem1):
    c = lax.axis_index("c")
    s = lax.axis_index("s")
    wid = c * _NS + s

    @pl.when(s < 10)
    def _init():
        pltpu.sync_copy(v_hbm.at[pl.ds(s * _SROWS, _SROWS)], stage_v)
        pltpu.sync_copy(stage_v, acc.at[pl.ds(s * _SROWS, _SROWS)])

    pltpu.sync_copy(src_hbm.at[wid], src_v)
    pltpu.sync_copy(dst_hbm.at[wid], dst_v)
    plsc.subcore_barrier()

    vbufs = (vals0_v, vals1_v, vals2_v, vals3_v)

    def gather(j, t):
        return pltpu.make_async_copy(v_hbm.at[src_v.at[j]],
                                     vbufs[t % 4], (sem0, sem1)[t % 2])

    gather(0, 0).start()
    gather(1, 1).start()

    def quad(q, carry):
        j0 = 4 * q
        for t in range(4):
            j = j0 + t
            gather(j, t).wait()
            pltpu.sync_copy(vbufs[t], acc.at[dst_v.at[j]], add=True)

            @pl.when(j + 2 < _STEPS)
            def _():
                gather(j + 2, t + 2).start()

        return carry

    lax.fori_loop(0, _STEPS // 4, quad, 0)
    plsc.subcore_barrier()

    @pl.when(s < 10)
    def _drain():
        pltpu.sync_copy(acc.at[pl.ds(s * _SROWS, _SROWS)], stage_v)
        pltpu.sync_copy(stage_v,
                        out_hbm.at[pl.ds(c * _N + s * _SROWS, _SROWS)])


_TCB = 2000
_TCG = _N // _TCB

_spec_half3 = pl.BlockSpec((_NC, _TCB, _HD), lambda i: (0, i, 0))
_spec_col = pl.BlockSpec((_TCB, 1), lambda i: (i, 0))
_spec_rows = pl.BlockSpec((_TCB, _D), lambda i: (i, 0))


def _tc_pre_body(cnt, x, w, dinv_ref, y_ref):
    deg = (cnt[:, 0] + cnt[:, 1] - 1.0)[:, None]
    dinv = lax.rsqrt(deg)
    dinv_ref[...] = dinv
    y_ref[...] = jnp.dot(x[...], w[...],
                         preferred_element_type=jnp.float32) * dinv


_tc_pre = pl.pallas_call(
    _tc_pre_body,
    grid=(_TCG,),
    in_specs=[
        pl.BlockSpec((_TCB, _NC), lambda i: (i, 0)),
        _spec_rows,
        pl.BlockSpec((_D, _D), lambda i: (0, 0)),
    ],
    out_specs=[_spec_col, _spec_rows],
    out_shape=[
        jax.ShapeDtypeStruct((_N, 1), jnp.float32),
        jax.ShapeDtypeStruct((_N, _D), jnp.float32),
    ],
)


def _tc_mid_body(p, y, dinv, b, w, out_ref):
    dv = dinv[...]
    agg = dv * (jnp.concatenate([p[0], p[1]], axis=1) + y[...]) + b[...]
    h = jnp.maximum(agg, 0.0)
    out_ref[...] = jnp.dot(h, w[...], preferred_element_type=jnp.float32) * dv


_tc_mid = pl.pallas_call(
    _tc_mid_body,
    grid=(_TCG,),
    in_specs=[
        _spec_half3,
        _spec_rows,
        _spec_col,
        pl.BlockSpec((1, _D), lambda i: (0, 0)),
        pl.BlockSpec((_D, _D), lambda i: (0, 0)),
    ],
    out_specs=_spec_rows,
    out_shape=jax.ShapeDtypeStruct((_N, _D), jnp.float32),
)


def _tc_fin_body(r, zs, dinv, b3, out_ref):
    out_ref[...] = dinv[...] * ((r[0] + r[1])[:, None] - zs[...]) + b3[...]


_tc_fin = pl.pallas_call(
    _tc_fin_body,
    out_shape=jax.ShapeDtypeStruct((_N, 1), jnp.float32),
)


def kernel(x, edge_index, W1, b1, W2, b2, W3, b3):
    src = edge_index[0].reshape(_NW, _STEPS, _CHUNK)
    dst = edge_index[1].reshape(_NW, _STEPS, _CHUNK)
    srcw = edge_index[0].reshape(_NS, _WSTEPS, _CHUNK)
    dstw = edge_index[1].reshape(_NS, _WSTEPS, _CHUNK)
    src2 = jnp.stack([srcw * 2, srcw * 2 + 1])
    ones = jnp.ones((_N,), jnp.float32)
    zeros = jnp.zeros((_N, _HD), jnp.float32)

    cnt = _sc_count(ones, dst).reshape(_NC, _N).T
    dinv, y1 = _tc_pre(cnt, x, W1)

    w3p = jnp.pad(W3, ((0, 0), (0, _D - 1)))

    def body(state):
        i, y, w_cur, w_nxt, b_cur, b_nxt = state
        p = _sc_agg(y.reshape(2 * _N, _HD), zeros, src2, dstw)
        y_nxt = _tc_mid(p, y, dinv, b_cur, w_cur)
        return i + 1, y_nxt, w_nxt, w_cur, b_nxt, b_cur

    i0 = (x[0, 0] - x[0, 0]).astype(jnp.int32)
    st = (i0, y1, W2, w3p, b1.reshape(1, _D), b2.reshape(1, _D))
    y3 = lax.while_loop(lambda s: s[0] < 2, body, st)[1]

    zs = y3[:, :1]
    r = _sc_sagg(y3[:, 0], src, dst).reshape(_NC, _N)
    out = _tc_fin(r, zs, dinv, b3.reshape(1, 1))
    return out
